# baseline (speedup 1.0000x reference)
import jax
import jax.numpy as jnp
from jax import lax
from jax.experimental import pallas as pl
from jax.experimental.pallas import tpu as pltpu
from jax.experimental.pallas import tpu_sc as plsc

_P = 8


def _sc_gather(emb, idx_flat):
    n = idx_flat.shape[0]
    d = emb.shape[1]
    window = 128
    mesh = plsc.VectorSubcoreMesh(core_axis_name="core", subcore_axis_name="subcore")
    idx2 = idx_flat.reshape(1, n)

    @pl.kernel(out_type=jax.ShapeDtypeStruct((n, d), emb.dtype), mesh=mesh)
    def gather_kernel(x_hbm, i_hbm, o_hbm):
        def body(i_vmem, o_vmem):
            pltpu.sync_copy(x_hbm.at[i_vmem.at[0]], o_vmem)

        pltpu.emit_pipeline(
            body,
            grid=(n // window,),
            in_specs=[pl.BlockSpec((1, window), index_map=lambda i: (0, i))],
            out_specs=[pl.BlockSpec((window, d), index_map=lambda i: (i, 0))],
            core_axis_name=("core", "subcore"),
            dimension_semantics=(pltpu.PARALLEL,),
        )(i_hbm, o_hbm)

    return gather_kernel(emb, idx2)


def _leaky_relu(x, alpha=0.2):
    return jnp.where(x >= 0, x, alpha * x)


_C00 = (((0,), (0,)), ((), ()))
_F32 = jnp.float32


def _layer_stacked(xs, xvs, masks, w2, w3, wc_row, a_lo_row, a_cat, a2_hi):
    p_docs = len(masks)
    n2, n1 = masks[0].shape
    x4 = jnp.dot(xs, w2, preferred_element_type=_F32)
    c1 = jnp.sum(wc_row * a_lo_row)
    su = jnp.dot(x4, a_cat, preferred_element_type=_F32)
    s1 = c1 + su[:, 0:1]
    e1 = _leaky_relu(s1)
    w1 = jnp.exp(e1 - jnp.max(e1))
    xw = xvs * w1
    ones_col = jnp.ones((n1, 1), _F32)
    nodes = []
    for p in range(p_docs):
        sl = slice(p * n2, (p + 1) * n2)
        mask_t = masks[p]
        edge_num = lax.dot_general(mask_t, xw[sl], _C00,
                                   preferred_element_type=_F32)
        den1 = lax.dot_general(mask_t, w1[sl], _C00,
                               preferred_element_type=_F32)
        inv1 = jnp.where(den1 > 0, 1.0 / den1, 0.0)
        mean_xv = jnp.sum(xvs[sl], axis=0, keepdims=True) * (1.0 / n2)
        edge = edge_num * inv1 + jnp.where(den1 > 0, 0.0, 1.0) * mean_xv
        edge4 = jnp.dot(edge, w3, preferred_element_type=_F32)
        u = su[sl, 1:2]
        v = jnp.dot(edge4, a2_hi, preferred_element_type=_F32)
        m2 = _leaky_relu(jnp.max(u) + jnp.max(v))
        s2t = u + v.reshape(1, n1)
        p2t = mask_t * jnp.exp(_leaky_relu(s2t) - m2)
        node_num = jnp.dot(p2t, edge, preferred_element_type=_F32)
        den2 = jnp.dot(p2t, ones_col, preferred_element_type=_F32)
        inv2 = jnp.where(den2 > 0, 1.0 / den2, 0.0)
        mean_edge = jnp.sum(edge, axis=0, keepdims=True) * (1.0 / n1)
        nodes.append(node_num * inv2 + jnp.where(den2 > 0, 0.0, 1.0) * mean_edge)
    return nodes


def _doc_kernel(x_ref, adj_ref,
                g1_w2_ref, g1_w3_ref, g1_wc_ref, g1_alo_ref, g1_acat_ref,
                g1_a2hi_ref,
                lin_w_ref, lin_b_ref,
                g2_w_ref, g2_w2_ref, g2_w3_ref, g2_wc_ref, g2_alo_ref,
                g2_acat_ref, g2_a2hi_ref,
                pred_w_ref, pred_b_ref, out_ref):
    n2, d = x_ref.shape[1], x_ref.shape[2]
    xs = x_ref[...].reshape(_P * n2, d)
    masks = [jnp.where(adj_ref[p] > 0.0, 1.0, 0.0).T for p in range(_P)]

    nodes1 = _layer_stacked(xs, xs, masks,
                            g1_w2_ref[...], g1_w3_ref[...], g1_wc_ref[...],
                            g1_alo_ref[...], g1_acat_ref[...], g1_a2hi_ref[...])
    node1 = jnp.concatenate(nodes1, axis=0)
    h = jnp.where(node1 > 0, node1, jnp.exp(jnp.minimum(node1, 0.0)) - 1.0)
    h = jnp.dot(h, lin_w_ref[...],
                preferred_element_type=_F32) + lin_b_ref[...]
    h = 0.5 * h * (1.0 + lax.erf(h * 0.7071067811865476))

    xv2 = jnp.dot(h, g2_w_ref[...], preferred_element_type=_F32)
    nodes2 = _layer_stacked(h, xv2, masks,
                            g2_w2_ref[...], g2_w3_ref[...], g2_wc_ref[...],
                            g2_alo_ref[...], g2_acat_ref[...], g2_a2hi_ref[...])

    for p in range(_P):
        pooled = jnp.sum(nodes2[p], axis=0, keepdims=True)
        logits = jnp.dot(pooled, pred_w_ref[...],
                         preferred_element_type=_F32) + pred_b_ref[...]
        m = jnp.max(logits, axis=1, keepdims=True)
        lse = m + jnp.log(jnp.sum(jnp.exp(logits - m), axis=1, keepdims=True))
        out_ref[p] = logits - lse


def _run_docs(xg, adj, weights):
    B, N2, D = xg.shape
    N1 = adj.shape[1]
    C = weights[-2].shape[1]

    def full(shape):
        return pl.BlockSpec(shape, lambda b: tuple(0 for _ in shape))

    in_specs = [
        pl.BlockSpec((_P, N2, D), lambda b: (b, 0, 0)),
        pl.BlockSpec((_P, N1, N2), lambda b: (b, 0, 0)),
    ] + [full(w.shape) for w in weights]

    return pl.pallas_call(
        _doc_kernel,
        grid=(B // _P,),
        in_specs=in_specs,
        out_specs=pl.BlockSpec((_P, 1, C), lambda b: (b, 0, 0)),
        out_shape=jax.ShapeDtypeStruct((B, 1, C), jnp.float32),
        compiler_params=pltpu.CompilerParams(
            dimension_semantics=("parallel",),
        ),
    )(xg, adj, *weights).reshape(B, C)


def kernel(words2ids, paris_mat, emb, g1_w2, g1_w3, g1_wc, g1_a, g1_a2,
           lin_w, lin_b, g2_w, g2_w2, g2_w3, g2_wc, g2_a, g2_a2,
           pred_w, pred_b):
    B, N2 = words2ids.shape
    D = emb.shape[1]
    H = lin_w.shape[1]

    idx = words2ids.reshape(-1).astype(jnp.int32)

    weights = (
        g1_w2, g1_w3,
        g1_wc.reshape(1, D), g1_a[:D].reshape(1, D),
        jnp.concatenate([g1_a[D:].reshape(D, 1), g1_a2[:D].reshape(D, 1)], axis=1),
        g1_a2[D:].reshape(D, 1),
        lin_w, lin_b.reshape(1, H),
        g2_w, g2_w2, g2_w3,
        g2_wc.reshape(1, H), g2_a[:H].reshape(1, H),
        jnp.concatenate([g2_a[H:].reshape(H, 1), g2_a2[:H].reshape(H, 1)], axis=1),
        g2_a2[H:].reshape(H, 1),
        pred_w, pred_b.reshape(1, pred_b.shape[0]),
    )
    half = B // 2
    n_half = half * N2
    xg0 = _sc_gather(emb, idx[:n_half]).reshape(half, N2, D)
    xg1 = _sc_gather(emb, idx[n_half:]).reshape(half, N2, D)
    out0 = _run_docs(xg0, paris_mat[:half], weights)
    out1 = _run_docs(xg1, paris_mat[half:], weights)
    return jnp.concatenate([out0, out1], axis=0)


# speedup vs baseline: 1.0136x; 1.0136x over previous
import jax
import jax.numpy as jnp
from jax import lax
from jax.experimental import pallas as pl
from jax.experimental.pallas import tpu as pltpu
from jax.experimental.pallas import tpu_sc as plsc

_P = 16


def _sc_gather(emb, idx_flat):
    n = idx_flat.shape[0]
    d = emb.shape[1]
    window = 128
    mesh = plsc.VectorSubcoreMesh(core_axis_name="core", subcore_axis_name="subcore")
    idx2 = idx_flat.reshape(1, n)

    @pl.kernel(out_type=jax.ShapeDtypeStruct((n, d), emb.dtype), mesh=mesh)
    def gather_kernel(x_hbm, i_hbm, o_hbm):
        def body(i_vmem, o_vmem):
            pltpu.sync_copy(x_hbm.at[i_vmem.at[0]], o_vmem)

        pltpu.emit_pipeline(
            body,
            grid=(n // window,),
            in_specs=[pl.BlockSpec((1, window), index_map=lambda i: (0, i))],
            out_specs=[pl.BlockSpec((window, d), index_map=lambda i: (i, 0))],
            core_axis_name=("core", "subcore"),
            dimension_semantics=(pltpu.PARALLEL,),
        )(i_hbm, o_hbm)

    return gather_kernel(emb, idx2)


def _leaky_relu(x, alpha=0.2):
    return jnp.where(x >= 0, x, alpha * x)


_C00 = (((0,), (0,)), ((), ()))
_F32 = jnp.float32


def _layer_stacked(xs, xvs, masks, w2, w3, wc_row, a_lo_row, a_cat, a2_hi):
    p_docs = len(masks)
    n2, n1 = masks[0].shape
    x4 = jnp.dot(xs, w2, preferred_element_type=_F32)
    c1 = jnp.sum(wc_row * a_lo_row)
    su = jnp.dot(x4, a_cat, preferred_element_type=_F32)
    s1 = c1 + su[:, 0:1]
    e1 = _leaky_relu(s1)
    w1 = jnp.exp(e1 - jnp.max(e1))
    xw = xvs * w1
    ones_col = jnp.ones((n1, 1), _F32)
    nodes = []
    for p in range(p_docs):
        sl = slice(p * n2, (p + 1) * n2)
        mask_t = masks[p]
        edge_num = lax.dot_general(mask_t, xw[sl], _C00,
                                   preferred_element_type=_F32)
        den1 = lax.dot_general(mask_t, w1[sl], _C00,
                               preferred_element_type=_F32)
        inv1 = jnp.where(den1 > 0, 1.0 / den1, 0.0)
        mean_xv = jnp.sum(xvs[sl], axis=0, keepdims=True) * (1.0 / n2)
        edge = edge_num * inv1 + jnp.where(den1 > 0, 0.0, 1.0) * mean_xv
        edge4 = jnp.dot(edge, w3, preferred_element_type=_F32)
        u = su[sl, 1:2]
        v = jnp.dot(edge4, a2_hi, preferred_element_type=_F32)
        m2 = _leaky_relu(jnp.max(u) + jnp.max(v))
        s2t = u + v.reshape(1, n1)
        p2t = mask_t * jnp.exp(_leaky_relu(s2t) - m2)
        node_num = jnp.dot(p2t, edge, preferred_element_type=_F32)
        den2 = jnp.dot(p2t, ones_col, preferred_element_type=_F32)
        inv2 = jnp.where(den2 > 0, 1.0 / den2, 0.0)
        mean_edge = jnp.sum(edge, axis=0, keepdims=True) * (1.0 / n1)
        nodes.append(node_num * inv2 + jnp.where(den2 > 0, 0.0, 1.0) * mean_edge)
    return nodes


def _doc_kernel(x_ref, adj_ref,
                g1_w2_ref, g1_w3_ref, g1_wc_ref, g1_alo_ref, g1_acat_ref,
                g1_a2hi_ref,
                lin_w_ref, lin_b_ref,
                g2_w_ref, g2_w2_ref, g2_w3_ref, g2_wc_ref, g2_alo_ref,
                g2_acat_ref, g2_a2hi_ref,
                pred_w_ref, pred_b_ref, out_ref):
    n2, d = x_ref.shape[1], x_ref.shape[2]
    xs = x_ref[...].reshape(_P * n2, d)
    masks = [jnp.where(adj_ref[p] > 0.0, 1.0, 0.0).T for p in range(_P)]

    nodes1 = _layer_stacked(xs, xs, masks,
                            g1_w2_ref[...], g1_w3_ref[...], g1_wc_ref[...],
                            g1_alo_ref[...], g1_acat_ref[...], g1_a2hi_ref[...])
    node1 = jnp.concatenate(nodes1, axis=0)
    h = jnp.where(node1 > 0, node1, jnp.exp(jnp.minimum(node1, 0.0)) - 1.0)
    h = jnp.dot(h, lin_w_ref[...],
                preferred_element_type=_F32) + lin_b_ref[...]
    h = 0.5 * h * (1.0 + lax.erf(h * 0.7071067811865476))

    xv2 = jnp.dot(h, g2_w_ref[...], preferred_element_type=_F32)
    nodes2 = _layer_stacked(h, xv2, masks,
                            g2_w2_ref[...], g2_w3_ref[...], g2_wc_ref[...],
                            g2_alo_ref[...], g2_acat_ref[...], g2_a2hi_ref[...])

    for p in range(_P):
        pooled = jnp.sum(nodes2[p], axis=0, keepdims=True)
        logits = jnp.dot(pooled, pred_w_ref[...],
                         preferred_element_type=_F32) + pred_b_ref[...]
        m = jnp.max(logits, axis=1, keepdims=True)
        lse = m + jnp.log(jnp.sum(jnp.exp(logits - m), axis=1, keepdims=True))
        out_ref[p] = logits - lse


def _run_docs(xg, adj, weights):
    B, N2, D = xg.shape
    N1 = adj.shape[1]
    C = weights[-2].shape[1]

    def full(shape):
        return pl.BlockSpec(shape, lambda b: tuple(0 for _ in shape))

    in_specs = [
        pl.BlockSpec((_P, N2, D), lambda b: (b, 0, 0)),
        pl.BlockSpec((_P, N1, N2), lambda b: (b, 0, 0)),
    ] + [full(w.shape) for w in weights]

    return pl.pallas_call(
        _doc_kernel,
        grid=(B // _P,),
        in_specs=in_specs,
        out_specs=pl.BlockSpec((_P, 1, C), lambda b: (b, 0, 0)),
        out_shape=jax.ShapeDtypeStruct((B, 1, C), jnp.float32),
        compiler_params=pltpu.CompilerParams(
            dimension_semantics=("parallel",),
        ),
    )(xg, adj, *weights).reshape(B, C)


def kernel(words2ids, paris_mat, emb, g1_w2, g1_w3, g1_wc, g1_a, g1_a2,
           lin_w, lin_b, g2_w, g2_w2, g2_w3, g2_wc, g2_a, g2_a2,
           pred_w, pred_b):
    B, N2 = words2ids.shape
    D = emb.shape[1]
    H = lin_w.shape[1]

    idx = words2ids.reshape(-1).astype(jnp.int32)

    weights = (
        g1_w2, g1_w3,
        g1_wc.reshape(1, D), g1_a[:D].reshape(1, D),
        jnp.concatenate([g1_a[D:].reshape(D, 1), g1_a2[:D].reshape(D, 1)], axis=1),
        g1_a2[D:].reshape(D, 1),
        lin_w, lin_b.reshape(1, H),
        g2_w, g2_w2, g2_w3,
        g2_wc.reshape(1, H), g2_a[:H].reshape(1, H),
        jnp.concatenate([g2_a[H:].reshape(H, 1), g2_a2[:H].reshape(H, 1)], axis=1),
        g2_a2[H:].reshape(H, 1),
        pred_w, pred_b.reshape(1, pred_b.shape[0]),
    )
    xg = _sc_gather(emb, idx).reshape(B, N2, D)
    return _run_docs(xg, paris_mat, weights)
